# baseline (speedup 1.0000x reference)
import functools

import jax
import jax.numpy as jnp
from jax import lax
from jax.experimental import pallas as pl
from jax.experimental.pallas import tpu as pltpu
from jax.experimental.pallas import tpu_sc as plsc

N = 10000
E = 320000
H = 128
OUT = 2
BN_EPS = 1e-5

NW = 32
CH = 128
NCHUNK = 80
PW = NCHUNK * CH
EP = NW * PW
NP = 10240
RPT = NP // 16

_mesh = plsc.VectorSubcoreMesh(core_axis_name="c", subcore_axis_name="s")


@functools.partial(
    pl.kernel,
    out_type=jax.ShapeDtypeStruct((2, NP, H), jnp.float32),
    mesh=_mesh,
    scratch_types=[
        pltpu.VMEM((NCHUNK, CH), jnp.int32),
        pltpu.VMEM((CH, H), jnp.float32),
        pltpu.VMEM_SHARED((NP, H), jnp.float32),
    ],
)
def _sc_count(dst_hbm, zeros_hbm, out_hbm, dst_v, ones_v, cnt_sh):
    c = lax.axis_index("c")
    s = lax.axis_index("s")
    wid = s * 2 + c
    pltpu.sync_copy(dst_hbm.at[wid], dst_v)
    ones16 = jnp.ones((16,), jnp.float32)

    def fbody(k, carry):
        r = k // 8
        col = (k % 8) * 16
        ones_v[r, pl.ds(col, 16)] = ones16
        return carry

    lax.fori_loop(0, CH * 8, fbody, 0)
    pltpu.sync_copy(zeros_hbm, cnt_sh.at[pl.ds(s * RPT, RPT)])
    plsc.subcore_barrier()

    def body(j, carry):
        pltpu.sync_copy(ones_v, cnt_sh.at[dst_v.at[j]], add=True)
        return carry

    lax.fori_loop(0, NCHUNK, body, 0)
    plsc.subcore_barrier()
    pltpu.sync_copy(cnt_sh.at[pl.ds(s * RPT, RPT)],
                    out_hbm.at[c, pl.ds(s * RPT, RPT)])


HALF = NCHUNK // 2


@functools.partial(
    pl.kernel,
    out_type=jax.ShapeDtypeStruct((2, NP, H), jnp.float32),
    mesh=_mesh,
    scratch_types=[
        pltpu.VMEM((HALF, CH), jnp.int32),
        pltpu.VMEM((HALF, CH), jnp.int32),
        pltpu.VMEM((CH, H), jnp.float32),
        pltpu.VMEM((CH, H), jnp.float32),
        pltpu.VMEM_SHARED((NP, H), jnp.float32),
        pltpu.SemaphoreType.DMA,
        pltpu.SemaphoreType.DMA,
    ],
)
def _sc_agg(h_hbm, src_hbm, dst_hbm, zeros_hbm, out_hbm,
            src_v, dst_v, buf0, buf1, agg_sh, sem0, sem1):
    c = lax.axis_index("c")
    s = lax.axis_index("s")
    wid = s * 2 + c
    pltpu.sync_copy(zeros_hbm, agg_sh.at[pl.ds(s * RPT, RPT)])
    plsc.subcore_barrier()

    for h0 in (0, HALF):
        pltpu.sync_copy(src_hbm.at[wid, pl.ds(h0, HALF)], src_v)
        pltpu.sync_copy(dst_hbm.at[wid, pl.ds(h0, HALF)], dst_v)
        pltpu.async_copy(h_hbm.at[src_v.at[0]], buf0, sem0)

        def body(i, carry):
            pltpu.async_copy(h_hbm.at[src_v.at[2 * i + 1]], buf1, sem1)
            pltpu.make_async_copy(h_hbm.at[src_v.at[0]], buf0, sem0).wait()
            pltpu.sync_copy(buf0, agg_sh.at[dst_v.at[2 * i]], add=True)

            @pl.when(i < HALF // 2 - 1)
            def _():
                pltpu.async_copy(h_hbm.at[src_v.at[2 * i + 2]], buf0, sem0)

            pltpu.make_async_copy(h_hbm.at[src_v.at[0]], buf1, sem1).wait()
            pltpu.sync_copy(buf1, agg_sh.at[dst_v.at[2 * i + 1]], add=True)
            return carry

        lax.fori_loop(0, HALF // 2, body, 0)
    plsc.subcore_barrier()
    pltpu.sync_copy(agg_sh.at[pl.ds(s * RPT, RPT)],
                    out_hbm.at[c, pl.ds(s * RPT, RPT)])


def _make_core_probe(which):
    @functools.partial(
        pl.kernel,
        out_type=jax.ShapeDtypeStruct((2, NP, H), jnp.float32),
        mesh=_mesh,
        scratch_types=[
            pltpu.VMEM((HALF, CH), jnp.int32),
            pltpu.VMEM((CH, H), jnp.float32),
            pltpu.VMEM((CH, H), jnp.float32),
            pltpu.VMEM_SHARED((NP, H), jnp.float32),
            pltpu.SemaphoreType.DMA,
            pltpu.SemaphoreType.DMA,
        ],
    )
    def _probe(h_hbm, src_hbm, dst_hbm, zeros_hbm, out_hbm,
               src_v, buf0, buf1, agg_sh, sem0, sem1):
        c = lax.axis_index("c")
        s = lax.axis_index("s")
        wid = s * 2 + c
        pltpu.sync_copy(zeros_hbm, agg_sh.at[pl.ds(s * RPT, RPT)])
        plsc.subcore_barrier()

        @pl.when(c == which)
        def _():
            for h0 in (0, HALF):
                pltpu.sync_copy(src_hbm.at[wid, pl.ds(h0, HALF)], src_v)

                def body(i, carry):
                    pltpu.async_copy(h_hbm.at[src_v.at[2 * i]], buf0, sem0)
                    pltpu.async_copy(h_hbm.at[src_v.at[2 * i + 1]], buf1, sem1)
                    pltpu.make_async_copy(h_hbm.at[src_v.at[0]], buf0,
                                          sem0).wait()
                    pltpu.make_async_copy(h_hbm.at[src_v.at[0]], buf1,
                                          sem1).wait()
                    return carry

                lax.fori_loop(0, HALF // 2, body, 0)

        plsc.subcore_barrier()
        pltpu.sync_copy(agg_sh.at[pl.ds(s * RPT, RPT)],
                        out_hbm.at[c, pl.ds(s * RPT, RPT)])
    return _probe


_sc_probe_c0 = _make_core_probe(0)
_sc_probe_c1 = _make_core_probe(1)


@functools.partial(
    pl.kernel,
    out_type=jax.ShapeDtypeStruct((2, NP, H), jnp.float32),
    mesh=_mesh,
    scratch_types=[
        pltpu.VMEM((HALF, CH), jnp.int32),
        pltpu.VMEM((HALF, CH), jnp.int32),
        pltpu.VMEM((CH, H), jnp.float32),
        pltpu.VMEM((CH, H), jnp.float32),
        pltpu.VMEM_SHARED((NP, H), jnp.float32),
        pltpu.SemaphoreType.DMA,
        pltpu.SemaphoreType.DMA,
    ],
)
def _sc_agg_gonly(h_hbm, src_hbm, dst_hbm, zeros_hbm, out_hbm,
                  src_v, dst_v, buf0, buf1, agg_sh, sem0, sem1):
    c = lax.axis_index("c")
    s = lax.axis_index("s")
    wid = s * 2 + c
    pltpu.sync_copy(zeros_hbm, agg_sh.at[pl.ds(s * RPT, RPT)])
    plsc.subcore_barrier()
    for h0 in (0, HALF):
        pltpu.sync_copy(src_hbm.at[wid, pl.ds(h0, HALF)], src_v)

        def body(i, carry):
            pltpu.async_copy(h_hbm.at[src_v.at[2 * i]], buf0, sem0)
            pltpu.async_copy(h_hbm.at[src_v.at[2 * i + 1]], buf1, sem1)
            pltpu.make_async_copy(h_hbm.at[src_v.at[0]], buf0, sem0).wait()
            pltpu.make_async_copy(h_hbm.at[src_v.at[0]], buf1, sem1).wait()
            return carry

        lax.fori_loop(0, HALF // 2, body, 0)
    plsc.subcore_barrier()
    pltpu.sync_copy(agg_sh.at[pl.ds(s * RPT, RPT)],
                    out_hbm.at[c, pl.ds(s * RPT, RPT)])


@functools.partial(
    pl.kernel,
    out_type=jax.ShapeDtypeStruct((2, NP, H), jnp.float32),
    mesh=_mesh,
    scratch_types=[
        pltpu.VMEM((HALF, CH), jnp.int32),
        pltpu.VMEM((HALF, CH), jnp.int32),
        pltpu.VMEM((CH, H), jnp.float32),
        pltpu.VMEM((CH, H), jnp.float32),
        pltpu.VMEM_SHARED((NP, H), jnp.float32),
        pltpu.SemaphoreType.DMA,
        pltpu.SemaphoreType.DMA,
    ],
)
def _sc_agg_sonly(h_hbm, src_hbm, dst_hbm, zeros_hbm, out_hbm,
                  src_v, dst_v, buf0, buf1, agg_sh, sem0, sem1):
    c = lax.axis_index("c")
    s = lax.axis_index("s")
    wid = s * 2 + c
    pltpu.sync_copy(zeros_hbm, agg_sh.at[pl.ds(s * RPT, RPT)])
    plsc.subcore_barrier()
    for h0 in (0, HALF):
        pltpu.sync_copy(dst_hbm.at[wid, pl.ds(h0, HALF)], dst_v)

        def body(i, carry):
            pltpu.sync_copy(buf0, agg_sh.at[dst_v.at[2 * i]], add=True)
            pltpu.sync_copy(buf1, agg_sh.at[dst_v.at[2 * i + 1]], add=True)
            return carry

        lax.fori_loop(0, HALF // 2, body, 0)
    plsc.subcore_barrier()
    pltpu.sync_copy(agg_sh.at[pl.ds(s * RPT, RPT)],
                    out_hbm.at[c, pl.ds(s * RPT, RPT)])


_RB = 2000
_GRID = N // _RB


def _proj_body(x_ref, w_ref, b_ref, o_ref):
    o_ref[...] = (jnp.dot(x_ref[...], w_ref[...],
                          preferred_element_type=jnp.float32) + b_ref[...])


def _tc_proj(x, w_t, b):
    return pl.pallas_call(
        _proj_body,
        grid=(_GRID,),
        in_specs=[
            pl.BlockSpec((_RB, H), lambda i: (i, 0)),
            pl.BlockSpec((H, H), lambda i: (0, 0)),
            pl.BlockSpec((1, H), lambda i: (0, 0)),
        ],
        out_specs=pl.BlockSpec((_RB, H), lambda i: (i, 0)),
        out_shape=jax.ShapeDtypeStruct((N, H), jnp.float32),
    )(x, w_t, b)


def _recip_body(parts_ref, o_ref):
    cnt = jnp.sum(parts_ref[...], axis=(0, 2)) * (1.0 / H)
    o_ref[...] = (1.0 / jnp.maximum(cnt, 1.0)).reshape(NP, 1)


def _tc_recip(parts):
    return pl.pallas_call(
        _recip_body,
        out_shape=jax.ShapeDtypeStruct((NP, 1), jnp.float32),
    )(parts)


def _layer_body(p_ref, r_ref, h_ref, wl_ref, bl_ref, wr_ref, sc_ref, sh_ref,
                o_ref):
    agg = (p_ref[0] + p_ref[1]) * r_ref[...]
    z = (jnp.dot(agg, wl_ref[...], preferred_element_type=jnp.float32)
         + jnp.dot(h_ref[...], wr_ref[...], preferred_element_type=jnp.float32)
         + bl_ref[...])
    o_ref[...] = jnp.maximum(z * sc_ref[...] + sh_ref[...], 0.0)


def _tc_layer(p, recip, h, wl_t, bl, wr_t, scale, shift):
    return pl.pallas_call(
        _layer_body,
        grid=(_GRID,),
        in_specs=[
            pl.BlockSpec((2, _RB, H), lambda i: (0, i, 0)),
            pl.BlockSpec((_RB, 1), lambda i: (i, 0)),
            pl.BlockSpec((_RB, H), lambda i: (i, 0)),
            pl.BlockSpec((H, H), lambda i: (0, 0)),
            pl.BlockSpec((1, H), lambda i: (0, 0)),
            pl.BlockSpec((H, H), lambda i: (0, 0)),
            pl.BlockSpec((1, H), lambda i: (0, 0)),
            pl.BlockSpec((1, H), lambda i: (0, 0)),
        ],
        out_specs=pl.BlockSpec((_RB, H), lambda i: (i, 0)),
        out_shape=jax.ShapeDtypeStruct((N, H), jnp.float32),
    )(p, recip, h, wl_t, bl, wr_t, scale, shift)


def _final_body(p_ref, r_ref, h_ref, wl_ref, bl_ref, wr_ref, sc_ref, sh_ref,
                wc1_ref, bc1_ref, wc2_ref, bc2_ref, o_ref, acc_sum, acc_max):
    i = pl.program_id(0)
    agg = (p_ref[0] + p_ref[1]) * r_ref[...]
    z = (jnp.dot(agg, wl_ref[...], preferred_element_type=jnp.float32)
         + jnp.dot(h_ref[...], wr_ref[...], preferred_element_type=jnp.float32)
         + bl_ref[...])
    hb = jnp.maximum(z * sc_ref[...] + sh_ref[...], 0.0)
    psum = jnp.sum(hb, axis=0, keepdims=True)
    pmax = jnp.max(hb, axis=0, keepdims=True)

    @pl.when(i == 0)
    def _():
        acc_sum[...] = psum
        acc_max[...] = pmax

    @pl.when(i > 0)
    def _():
        acc_sum[...] = acc_sum[...] + psum
        acc_max[...] = jnp.maximum(acc_max[...], pmax)

    @pl.when(i == _GRID - 1)
    def _():
        mean = acc_sum[...] * (1.0 / N)
        rep = jnp.concatenate([mean, acc_max[...]], axis=1)
        zz = jnp.maximum(
            jnp.dot(rep, wc1_ref[...], preferred_element_type=jnp.float32)
            + bc1_ref[...], 0.0)
        o_ref[...] = (jnp.dot(zz, wc2_ref[...],
                              preferred_element_type=jnp.float32)
                      + bc2_ref[...])


def _tc_final(p, recip, h, wl_t, bl, wr_t, scale, shift, wc1_t, bc1, wc2_t,
              bc2):
    return pl.pallas_call(
        _final_body,
        grid=(_GRID,),
        in_specs=[
            pl.BlockSpec((2, _RB, H), lambda i: (0, i, 0)),
            pl.BlockSpec((_RB, 1), lambda i: (i, 0)),
            pl.BlockSpec((_RB, H), lambda i: (i, 0)),
            pl.BlockSpec((H, H), lambda i: (0, 0)),
            pl.BlockSpec((1, H), lambda i: (0, 0)),
            pl.BlockSpec((H, H), lambda i: (0, 0)),
            pl.BlockSpec((1, H), lambda i: (0, 0)),
            pl.BlockSpec((1, H), lambda i: (0, 0)),
            pl.BlockSpec((2 * H, H), lambda i: (0, 0)),
            pl.BlockSpec((1, H), lambda i: (0, 0)),
            pl.BlockSpec((H, OUT), lambda i: (0, 0)),
            pl.BlockSpec((1, OUT), lambda i: (0, 0)),
        ],
        out_specs=pl.BlockSpec((1, OUT), lambda i: (0, 0)),
        out_shape=jax.ShapeDtypeStruct((1, OUT), jnp.float32),
        scratch_shapes=[
            pltpu.VMEM((1, H), jnp.float32),
            pltpu.VMEM((1, H), jnp.float32),
        ],
    )(p, recip, h, wl_t, bl, wr_t, scale, shift, wc1_t, bc1, wc2_t, bc2)


def kernel(x, edge_index, W_in, b_in,
           Wl0, bl0, Wr0, g0, be0,
           Wl1, bl1, Wr1, g1, be1,
           Wl2, bl2, Wr2, g2, be2,
           Wc1, bc1, Wc2, bc2):
    pad = EP - E
    src_p = jnp.concatenate(
        [edge_index[0], jnp.zeros((pad,), jnp.int32)]).reshape(NW, NCHUNK, CH)
    dst_p = jnp.concatenate(
        [edge_index[1], jnp.full((pad,), N, jnp.int32)]).reshape(NW, NCHUNK, CH)
    zeros_rows = jnp.zeros((RPT, H), jnp.float32)

    cnt_parts = _sc_count(dst_p, zeros_rows)
    recip = _tc_recip(cnt_parts)

    bn = 1.0 / jnp.sqrt(jnp.float32(1.0 + BN_EPS))
    h = _tc_proj(x, W_in.T, b_in.reshape(1, H))
    def _agg(hh):
        return _sc_agg(hh, src_p, dst_p, zeros_rows)

    first = True
    for (Wl, bl, Wr, g, be) in ((Wl0, bl0, Wr0, g0, be0),
                                (Wl1, bl1, Wr1, g1, be1)):
        if first:
            p = (_sc_probe_c0(h, src_p, dst_p, zeros_rows)
                 + _sc_probe_c1(h, src_p, dst_p, zeros_rows))
            first = False
        else:
            p = _agg(h)
        h = _tc_layer(p, recip, h, Wl.T, bl.reshape(1, H), Wr.T,
                      (g * bn).reshape(1, H), be.reshape(1, H))
    p = _agg(h)
    logits = _tc_final(p, recip, h, Wl2.T, bl2.reshape(1, H), Wr2.T,
                       (g2 * bn).reshape(1, H), be2.reshape(1, H),
                       Wc1.T, bc1.reshape(1, H), Wc2.T, bc2.reshape(1, OUT))
    return logits


# speedup vs baseline: 1.2978x; 1.2978x over previous
import functools

import jax
import jax.numpy as jnp
from jax import lax
from jax.experimental import pallas as pl
from jax.experimental.pallas import tpu as pltpu
from jax.experimental.pallas import tpu_sc as plsc

N = 10000
E = 320000
H = 128
OUT = 2
BN_EPS = 1e-5

NW = 32
CH = 128
NCHUNK = 80
PW = NCHUNK * CH
EP = NW * PW
NP = 10240
RPT = NP // 16

_mesh = plsc.VectorSubcoreMesh(core_axis_name="c", subcore_axis_name="s")


@functools.partial(
    pl.kernel,
    out_type=jax.ShapeDtypeStruct((2, NP, H), jnp.float32),
    mesh=_mesh,
    scratch_types=[
        pltpu.VMEM((NCHUNK, CH), jnp.int32),
        pltpu.VMEM((CH, H), jnp.float32),
        pltpu.VMEM_SHARED((NP, H), jnp.float32),
    ],
)
def _sc_count(dst_hbm, zeros_hbm, out_hbm, dst_v, ones_v, cnt_sh):
    c = lax.axis_index("c")
    s = lax.axis_index("s")
    wid = s * 2 + c
    pltpu.sync_copy(dst_hbm.at[wid], dst_v)
    ones16 = jnp.ones((16,), jnp.float32)

    def fbody(k, carry):
        r = k // 8
        col = (k % 8) * 16
        ones_v[r, pl.ds(col, 16)] = ones16
        return carry

    lax.fori_loop(0, CH * 8, fbody, 0)
    pltpu.sync_copy(zeros_hbm, cnt_sh.at[pl.ds(s * RPT, RPT)])
    plsc.subcore_barrier()

    def body(j, carry):
        pltpu.sync_copy(ones_v, cnt_sh.at[dst_v.at[j]], add=True)
        return carry

    lax.fori_loop(0, NCHUNK, body, 0)
    plsc.subcore_barrier()
    pltpu.sync_copy(cnt_sh.at[pl.ds(s * RPT, RPT)],
                    out_hbm.at[c, pl.ds(s * RPT, RPT)])


NCHUNKS_TOT = NW * NCHUNK
NC0 = 128
NC1 = 32
ST = 32
assert 16 * (NC0 + NC1) == NCHUNKS_TOT and NC0 % ST == 0 and NC1 % ST == 0


@functools.partial(
    pl.kernel,
    out_type=jax.ShapeDtypeStruct((2, NP, H), jnp.float32),
    mesh=_mesh,
    scratch_types=[
        pltpu.VMEM((ST, CH), jnp.int32),
        pltpu.VMEM((ST, CH), jnp.int32),
        pltpu.VMEM((CH, H), jnp.float32),
        pltpu.VMEM((CH, H), jnp.float32),
        pltpu.VMEM_SHARED((NP, H), jnp.float32),
        pltpu.SemaphoreType.DMA,
        pltpu.SemaphoreType.DMA,
    ],
)
def _sc_agg(h_hbm, src_hbm, dst_hbm, zeros_hbm, out_hbm,
            src_v, dst_v, buf0, buf1, agg_sh, sem0, sem1):
    c = lax.axis_index("c")
    s = lax.axis_index("s")
    pltpu.sync_copy(zeros_hbm, agg_sh.at[pl.ds(s * RPT, RPT)])
    plsc.subcore_barrier()

    def _stage(base, L):
        pltpu.sync_copy(src_hbm.at[pl.ds(base, L)], src_v)
        pltpu.sync_copy(dst_hbm.at[pl.ds(base, L)], dst_v)
        pltpu.async_copy(h_hbm.at[src_v.at[0]], buf0, sem0)

        def body(i, carry):
            pltpu.async_copy(h_hbm.at[src_v.at[2 * i + 1]], buf1, sem1)
            pltpu.make_async_copy(h_hbm.at[src_v.at[0]], buf0, sem0).wait()
            pltpu.sync_copy(buf0, agg_sh.at[dst_v.at[2 * i]], add=True)

            @pl.when(i < L // 2 - 1)
            def _():
                pltpu.async_copy(h_hbm.at[src_v.at[2 * i + 2]], buf0, sem0)

            pltpu.make_async_copy(h_hbm.at[src_v.at[0]], buf1, sem1).wait()
            pltpu.sync_copy(buf1, agg_sh.at[dst_v.at[2 * i + 1]], add=True)
            return carry

        lax.fori_loop(0, L // 2, body, 0)

    @pl.when(c == 0)
    def _():
        for st in range(NC0 // ST):
            _stage(s * NC0 + st * ST, ST)

    @pl.when(c == 1)
    def _():
        _stage(16 * NC0 + s * NC1, NC1)

    plsc.subcore_barrier()
    pltpu.sync_copy(agg_sh.at[pl.ds(s * RPT, RPT)],
                    out_hbm.at[c, pl.ds(s * RPT, RPT)])


_RB = 2000
_GRID = N // _RB


def _proj_body(x_ref, w_ref, b_ref, o_ref):
    o_ref[...] = (jnp.dot(x_ref[...], w_ref[...],
                          preferred_element_type=jnp.float32) + b_ref[...])


def _tc_proj(x, w_t, b):
    return pl.pallas_call(
        _proj_body,
        grid=(_GRID,),
        in_specs=[
            pl.BlockSpec((_RB, H), lambda i: (i, 0)),
            pl.BlockSpec((H, H), lambda i: (0, 0)),
            pl.BlockSpec((1, H), lambda i: (0, 0)),
        ],
        out_specs=pl.BlockSpec((_RB, H), lambda i: (i, 0)),
        out_shape=jax.ShapeDtypeStruct((N, H), jnp.float32),
    )(x, w_t, b)


def _recip_body(parts_ref, o_ref):
    cnt = jnp.sum(parts_ref[...], axis=(0, 2)) * (1.0 / H)
    o_ref[...] = (1.0 / jnp.maximum(cnt, 1.0)).reshape(NP, 1)


def _tc_recip(parts):
    return pl.pallas_call(
        _recip_body,
        out_shape=jax.ShapeDtypeStruct((NP, 1), jnp.float32),
    )(parts)


def _layer_body(p_ref, r_ref, h_ref, wl_ref, bl_ref, wr_ref, sc_ref, sh_ref,
                o_ref):
    agg = (p_ref[0] + p_ref[1]) * r_ref[...]
    z = (jnp.dot(agg, wl_ref[...], preferred_element_type=jnp.float32)
         + jnp.dot(h_ref[...], wr_ref[...], preferred_element_type=jnp.float32)
         + bl_ref[...])
    o_ref[...] = jnp.maximum(z * sc_ref[...] + sh_ref[...], 0.0)


def _tc_layer(p, recip, h, wl_t, bl, wr_t, scale, shift):
    return pl.pallas_call(
        _layer_body,
        grid=(_GRID,),
        in_specs=[
            pl.BlockSpec((2, _RB, H), lambda i: (0, i, 0)),
            pl.BlockSpec((_RB, 1), lambda i: (i, 0)),
            pl.BlockSpec((_RB, H), lambda i: (i, 0)),
            pl.BlockSpec((H, H), lambda i: (0, 0)),
            pl.BlockSpec((1, H), lambda i: (0, 0)),
            pl.BlockSpec((H, H), lambda i: (0, 0)),
            pl.BlockSpec((1, H), lambda i: (0, 0)),
            pl.BlockSpec((1, H), lambda i: (0, 0)),
        ],
        out_specs=pl.BlockSpec((_RB, H), lambda i: (i, 0)),
        out_shape=jax.ShapeDtypeStruct((N, H), jnp.float32),
    )(p, recip, h, wl_t, bl, wr_t, scale, shift)


def _final_body(p_ref, r_ref, h_ref, wl_ref, bl_ref, wr_ref, sc_ref, sh_ref,
                wc1_ref, bc1_ref, wc2_ref, bc2_ref, o_ref, acc_sum, acc_max):
    i = pl.program_id(0)
    agg = (p_ref[0] + p_ref[1]) * r_ref[...]
    z = (jnp.dot(agg, wl_ref[...], preferred_element_type=jnp.float32)
         + jnp.dot(h_ref[...], wr_ref[...], preferred_element_type=jnp.float32)
         + bl_ref[...])
    hb = jnp.maximum(z * sc_ref[...] + sh_ref[...], 0.0)
    psum = jnp.sum(hb, axis=0, keepdims=True)
    pmax = jnp.max(hb, axis=0, keepdims=True)

    @pl.when(i == 0)
    def _():
        acc_sum[...] = psum
        acc_max[...] = pmax

    @pl.when(i > 0)
    def _():
        acc_sum[...] = acc_sum[...] + psum
        acc_max[...] = jnp.maximum(acc_max[...], pmax)

    @pl.when(i == _GRID - 1)
    def _():
        mean = acc_sum[...] * (1.0 / N)
        rep = jnp.concatenate([mean, acc_max[...]], axis=1)
        zz = jnp.maximum(
            jnp.dot(rep, wc1_ref[...], preferred_element_type=jnp.float32)
            + bc1_ref[...], 0.0)
        o_ref[...] = (jnp.dot(zz, wc2_ref[...],
                              preferred_element_type=jnp.float32)
                      + bc2_ref[...])


def _tc_final(p, recip, h, wl_t, bl, wr_t, scale, shift, wc1_t, bc1, wc2_t,
              bc2):
    return pl.pallas_call(
        _final_body,
        grid=(_GRID,),
        in_specs=[
            pl.BlockSpec((2, _RB, H), lambda i: (0, i, 0)),
            pl.BlockSpec((_RB, 1), lambda i: (i, 0)),
            pl.BlockSpec((_RB, H), lambda i: (i, 0)),
            pl.BlockSpec((H, H), lambda i: (0, 0)),
            pl.BlockSpec((1, H), lambda i: (0, 0)),
            pl.BlockSpec((H, H), lambda i: (0, 0)),
            pl.BlockSpec((1, H), lambda i: (0, 0)),
            pl.BlockSpec((1, H), lambda i: (0, 0)),
            pl.BlockSpec((2 * H, H), lambda i: (0, 0)),
            pl.BlockSpec((1, H), lambda i: (0, 0)),
            pl.BlockSpec((H, OUT), lambda i: (0, 0)),
            pl.BlockSpec((1, OUT), lambda i: (0, 0)),
        ],
        out_specs=pl.BlockSpec((1, OUT), lambda i: (0, 0)),
        out_shape=jax.ShapeDtypeStruct((1, OUT), jnp.float32),
        scratch_shapes=[
            pltpu.VMEM((1, H), jnp.float32),
            pltpu.VMEM((1, H), jnp.float32),
        ],
    )(p, recip, h, wl_t, bl, wr_t, scale, shift, wc1_t, bc1, wc2_t, bc2)


def kernel(x, edge_index, W_in, b_in,
           Wl0, bl0, Wr0, g0, be0,
           Wl1, bl1, Wr1, g1, be1,
           Wl2, bl2, Wr2, g2, be2,
           Wc1, bc1, Wc2, bc2):
    pad = EP - E
    src_p = jnp.concatenate(
        [edge_index[0], jnp.zeros((pad,), jnp.int32)]).reshape(NW, NCHUNK, CH)
    dst_p = jnp.concatenate(
        [edge_index[1], jnp.full((pad,), N, jnp.int32)]).reshape(NW, NCHUNK, CH)
    zeros_rows = jnp.zeros((RPT, H), jnp.float32)

    cnt_parts = _sc_count(dst_p, zeros_rows)
    recip = _tc_recip(cnt_parts)

    bn = 1.0 / jnp.sqrt(jnp.float32(1.0 + BN_EPS))
    h = _tc_proj(x, W_in.T, b_in.reshape(1, H))
    src_flat = src_p.reshape(NW * NCHUNK, CH)
    dst_flat = dst_p.reshape(NW * NCHUNK, CH)

    def _agg(hh):
        return _sc_agg(hh, src_flat, dst_flat, zeros_rows)

    for (Wl, bl, Wr, g, be) in ((Wl0, bl0, Wr0, g0, be0),
                                (Wl1, bl1, Wr1, g1, be1)):
        p = _agg(h)
        h = _tc_layer(p, recip, h, Wl.T, bl.reshape(1, H), Wr.T,
                      (g * bn).reshape(1, H), be.reshape(1, H))
    p = _agg(h)
    logits = _tc_final(p, recip, h, Wl2.T, bl2.reshape(1, H), Wr2.T,
                       (g2 * bn).reshape(1, H), be2.reshape(1, H),
                       Wc1.T, bc1.reshape(1, H), Wc2.T, bc2.reshape(1, OUT))
    return logits
